# Initial kernel scaffold; baseline (speedup 1.0000x reference)
import functools

import jax
import jax.numpy as jnp
from jax import lax
from jax.experimental import pallas as pl
from jax.experimental.pallas import tpu as pltpu
from jax.experimental.pallas import tpu_sc as plsc

N = 10000
E = 320000
D = 128
HID = 128
OUT = 64
H = 2

NW = 32
EP = E // NW
C = 80
NCH = EP // C
RP = N // 16


@functools.cache
def _make_sc_edge(Hc):
    HcP = Hc + 16
    mesh = plsc.VectorSubcoreMesh(core_axis_name="c", subcore_axis_name="s")

    @functools.partial(
        pl.kernel,
        out_type=jax.ShapeDtypeStruct((2, N, HcP), jnp.float32),
        mesh=mesh,
        scratch_types=[
            pltpu.VMEM_SHARED((N, HcP), jnp.float32),
            pltpu.VMEM((2 * N,), jnp.float32),
            pltpu.VMEM((2 * N,), jnp.float32),
            pltpu.VMEM((C,), jnp.int32),
            pltpu.VMEM((C,), jnp.int32),
            pltpu.VMEM((C, Hc), jnp.float32),
            pltpu.VMEM((C, HcP), jnp.float32),
            pltpu.VMEM((C,), jnp.float32),
            pltpu.VMEM((C,), jnp.float32),
            pltpu.SemaphoreType.DMA,
        ],
    )
    def sc_edge(hs_hbm, als_hbm, ald_hbm, src_hbm, dst_hbm, zeros_hbm, out_hbm,
                acc_sh, als_v, ald_v, sidx_v, didx_v, rows_v, msg_v,
                exa_v, exb_v, sem):
        cid = lax.axis_index("c")
        sid = lax.axis_index("s")
        wid = cid * 16 + sid

        pltpu.sync_copy(als_hbm, als_v)
        pltpu.sync_copy(ald_hbm, ald_v)
        pltpu.sync_copy(zeros_hbm.at[pl.ds(sid * RP, RP)],
                        acc_sh.at[pl.ds(sid * RP, RP)])
        plsc.subcore_barrier()

        base = wid * EP

        def chunk(ch, carry):
            off = base + ch * C
            pltpu.sync_copy(src_hbm.at[pl.ds(off, C)], sidx_v)
            pltpu.sync_copy(dst_hbm.at[pl.ds(off, C)], didx_v)
            gather = pltpu.async_copy(hs_hbm.at[sidx_v], rows_v, sem)
            for g in range(C // 16):
                s16 = sidx_v[pl.ds(g * 16, 16)] * 2
                d16 = didx_v[pl.ds(g * 16, 16)] * 2
                a0 = plsc.load_gather(als_v, [s16]) + plsc.load_gather(ald_v, [d16])
                a1 = (plsc.load_gather(als_v, [s16 + 1])
                      + plsc.load_gather(ald_v, [d16 + 1]))
                a0 = jnp.where(a0 >= 0, a0, 0.2 * a0)
                a1 = jnp.where(a1 >= 0, a1, 0.2 * a1)
                exa_v[pl.ds(g * 16, 16)] = jnp.exp(a0)
                exb_v[pl.ds(g * 16, 16)] = jnp.exp(a1)
            gather.wait()

            iota = lax.iota(jnp.int32, 16)

            def edge(e, carry2):
                e0 = exa_v[e]
                e1 = exb_v[e]
                for j in range(Hc // 16):
                    sc = e0 if j < Hc // 32 else e1
                    msg_v[e, pl.ds(j * 16, 16)] = rows_v[e, pl.ds(j * 16, 16)] * sc
                tail = jnp.where(iota == 0, e0, 0.0) + jnp.where(iota == 1, e1, 0.0)
                msg_v[e, pl.ds(Hc, 16)] = tail
                return carry2

            lax.fori_loop(0, C, edge, 0)
            pltpu.sync_copy(msg_v, acc_sh.at[didx_v], add=True)
            return carry

        lax.fori_loop(0, NCH, chunk, 0)
        plsc.subcore_barrier()
        pltpu.sync_copy(acc_sh.at[pl.ds(sid * RP, RP)],
                        out_hbm.at[cid, pl.ds(sid * RP, RP)])

    return sc_edge


_GRID = 10
_B = N // _GRID


def _row_spec(cols):
    return pl.BlockSpec((_B, cols), lambda i: (i, 0))


def _full_spec(r, cols):
    return pl.BlockSpec((r, cols), lambda i: (0, 0))


def _logits(hs, a_ref, al_ref, c):
    for h in range(H):
        al_ref[:, h:h + 1] = jnp.sum(
            hs[:, h * c:(h + 1) * c] * a_ref[h:h + 1, :], axis=1, keepdims=True)


def _tc1_body(xp_ref, xa_ref,
              wsc, wdc, asc, adc, wsw, wdw, asw, adw, wsr, wdr, asr, adr,
              hsc_o, alsc_o, aldc_o, hsw_o, alsw_o, aldw_o,
              hsr_o, alsr_o, aldr_o):
    xp = xp_ref[...]
    xa = xa_ref[...]
    c = HID // H
    for xs, xd, ws, wd, a_s, a_d, hs_o, als_o, ald_o in (
            (xp, xp, wsc, wdc, asc, adc, hsc_o, alsc_o, aldc_o),
            (xa, xp, wsw, wdw, asw, adw, hsw_o, alsw_o, aldw_o),
            (xp, xa, wsr, wdr, asr, adr, hsr_o, alsr_o, aldr_o)):
        hs = jnp.dot(xs, ws[...], preferred_element_type=jnp.float32)
        hs_o[...] = hs
        _logits(hs, a_s, als_o, c)
        hd = jnp.dot(xd, wd[...], preferred_element_type=jnp.float32)
        _logits(hd, a_d, ald_o, c)


def _combine(acc_ref, hc):
    num = acc_ref[0, :, :] + acc_ref[1, :, :]
    c = hc // H
    d0 = num[:, hc:hc + 1] + 1e-16
    d1 = num[:, hc + 1:hc + 2] + 1e-16
    return jnp.concatenate([num[:, :c] / d0, num[:, c:hc] / d1], axis=1)


def _tc2_body(accc, accw, accr, b1c, b1w, b1r,
              wsc, wdc, asc, adc, wsw, wdw, asw, adw, wsr, wdr, asr, adr,
              hsc_o, alsc_o, aldc_o, hsw_o, alsw_o, aldw_o,
              hsr_o, alsr_o, aldr_o):
    hp = jax.nn.relu(_combine(accc, HID) + _combine(accw, HID)
                     + b1c[...] + b1w[...])
    ha = jax.nn.relu(_combine(accr, HID) + b1r[...])
    c = OUT // H
    for xs, xd, ws, wd, a_s, a_d, hs_o, als_o, ald_o in (
            (hp, hp, wsc, wdc, asc, adc, hsc_o, alsc_o, aldc_o),
            (ha, hp, wsw, wdw, asw, adw, hsw_o, alsw_o, aldw_o),
            (hp, ha, wsr, wdr, asr, adr, hsr_o, alsr_o, aldr_o)):
        hs = jnp.dot(xs, ws[...], preferred_element_type=jnp.float32)
        hs_o[...] = hs
        _logits(hs, a_s, als_o, c)
        hd = jnp.dot(xd, wd[...], preferred_element_type=jnp.float32)
        _logits(hd, a_d, ald_o, c)


def _tc3_body(accc, accw, accr, b2c, b2w, b2r, op_o, oa_o):
    op_o[...] = _combine(accc, OUT) + _combine(accw, OUT) + b2c[...] + b2w[...]
    oa_o[...] = _combine(accr, OUT) + b2r[...]


def _acc_spec(hcp):
    return pl.BlockSpec((2, _B, hcp), lambda i: (0, i, 0))


def kernel(x_paper, x_author, edge_index_cites, edge_index_writes,
           edge_index_rev_writes,
           l1_cites_Wsrc, l1_cites_Wdst, l1_cites_asrc, l1_cites_adst, l1_cites_b,
           l1_writes_Wsrc, l1_writes_Wdst, l1_writes_asrc, l1_writes_adst, l1_writes_b,
           l1_rev_writes_Wsrc, l1_rev_writes_Wdst, l1_rev_writes_asrc,
           l1_rev_writes_adst, l1_rev_writes_b,
           l2_cites_Wsrc, l2_cites_Wdst, l2_cites_asrc, l2_cites_adst, l2_cites_b,
           l2_writes_Wsrc, l2_writes_Wdst, l2_writes_asrc, l2_writes_adst, l2_writes_b,
           l2_rev_writes_Wsrc, l2_rev_writes_Wdst, l2_rev_writes_asrc,
           l2_rev_writes_adst, l2_rev_writes_b):
    f32 = jnp.float32
    ei_c = edge_index_cites.astype(jnp.int32)
    ei_w = edge_index_writes.astype(jnp.int32)
    ei_r = edge_index_rev_writes.astype(jnp.int32)

    row128 = jax.ShapeDtypeStruct((N, HID), f32)
    al2 = jax.ShapeDtypeStruct((N, H), f32)
    tc1 = pl.pallas_call(
        _tc1_body,
        grid=(_GRID,),
        in_specs=[_row_spec(D), _row_spec(D)] + [
            s for _ in range(3) for s in
            (_full_spec(D, HID), _full_spec(D, HID),
             _full_spec(H, HID // H), _full_spec(H, HID // H))],
        out_specs=[s for _ in range(3) for s in
                   (_row_spec(HID), _row_spec(H), _row_spec(H))],
        out_shape=[s for _ in range(3) for s in (row128, al2, al2)],
    )
    (hs_c, als_c, ald_c, hs_w, als_w, ald_w, hs_r, als_r, ald_r) = tc1(
        x_paper, x_author,
        l1_cites_Wsrc, l1_cites_Wdst, l1_cites_asrc, l1_cites_adst,
        l1_writes_Wsrc, l1_writes_Wdst, l1_writes_asrc, l1_writes_adst,
        l1_rev_writes_Wsrc, l1_rev_writes_Wdst, l1_rev_writes_asrc,
        l1_rev_writes_adst)

    sc1 = _make_sc_edge(HID)
    zeros1 = jnp.zeros((N, HID + 16), f32)
    acc_c = sc1(hs_c, als_c.reshape(-1), ald_c.reshape(-1),
                ei_c[0], ei_c[1], zeros1)
    acc_w = sc1(hs_w, als_w.reshape(-1), ald_w.reshape(-1),
                ei_w[0], ei_w[1], zeros1)
    acc_r = sc1(hs_r, als_r.reshape(-1), ald_r.reshape(-1),
                ei_r[0], ei_r[1], zeros1)

    row64 = jax.ShapeDtypeStruct((N, OUT), f32)
    tc2 = pl.pallas_call(
        _tc2_body,
        grid=(_GRID,),
        in_specs=[_acc_spec(HID + 16)] * 3 + [_full_spec(1, HID)] * 3 + [
            s for _ in range(3) for s in
            (_full_spec(HID, OUT), _full_spec(HID, OUT),
             _full_spec(H, OUT // H), _full_spec(H, OUT // H))],
        out_specs=[s for _ in range(3) for s in
                   (_row_spec(OUT), _row_spec(H), _row_spec(H))],
        out_shape=[s for _ in range(3) for s in (row64, al2, al2)],
    )
    (hs2_c, als2_c, ald2_c, hs2_w, als2_w, ald2_w, hs2_r, als2_r, ald2_r) = tc2(
        acc_c, acc_w, acc_r,
        l1_cites_b.reshape(1, HID), l1_writes_b.reshape(1, HID),
        l1_rev_writes_b.reshape(1, HID),
        l2_cites_Wsrc, l2_cites_Wdst, l2_cites_asrc, l2_cites_adst,
        l2_writes_Wsrc, l2_writes_Wdst, l2_writes_asrc, l2_writes_adst,
        l2_rev_writes_Wsrc, l2_rev_writes_Wdst, l2_rev_writes_asrc,
        l2_rev_writes_adst)

    sc2 = _make_sc_edge(OUT)
    zeros2 = jnp.zeros((N, OUT + 16), f32)
    acc2_c = sc2(hs2_c, als2_c.reshape(-1), ald2_c.reshape(-1),
                 ei_c[0], ei_c[1], zeros2)
    acc2_w = sc2(hs2_w, als2_w.reshape(-1), ald2_w.reshape(-1),
                 ei_w[0], ei_w[1], zeros2)
    acc2_r = sc2(hs2_r, als2_r.reshape(-1), ald2_r.reshape(-1),
                 ei_r[0], ei_r[1], zeros2)

    tc3 = pl.pallas_call(
        _tc3_body,
        grid=(_GRID,),
        in_specs=[_acc_spec(OUT + 16)] * 3 + [_full_spec(1, OUT)] * 3,
        out_specs=[_row_spec(OUT), _row_spec(OUT)],
        out_shape=[row64, row64],
    )
    op, oa = tc3(acc2_c, acc2_w, acc2_r,
                 l2_cites_b.reshape(1, OUT), l2_writes_b.reshape(1, OUT),
                 l2_rev_writes_b.reshape(1, OUT))
    return op, oa


# speedup vs baseline: 65.7531x; 65.7531x over previous
import functools

import jax
import jax.numpy as jnp
from jax import lax
from jax.experimental import pallas as pl
from jax.experimental.pallas import tpu as pltpu
from jax.experimental.pallas import tpu_sc as plsc

N = 10000
E = 320000
D = 128
HID = 128
OUT = 64
H = 2

NW = 32
EP = E // NW
C = 80
NCH = EP // C
RP = N // 16


@functools.cache
def _make_sc_edge(Hc):
    HcP = Hc + 16
    mesh = plsc.VectorSubcoreMesh(core_axis_name="c", subcore_axis_name="s")

    @functools.partial(
        pl.kernel,
        out_type=jax.ShapeDtypeStruct((2, N, HcP), jnp.float32),
        mesh=mesh,
        compiler_params=pltpu.CompilerParams(use_tc_tiling_on_sc=False,
                                             needs_layout_passes=False),
        scratch_types=[
            pltpu.VMEM_SHARED((N, HcP), jnp.float32),
            pltpu.VMEM((2 * N,), jnp.float32),
            pltpu.VMEM((C,), jnp.int32),
            pltpu.VMEM((C,), jnp.int32),
            pltpu.VMEM((C, HcP), jnp.float32),
            pltpu.SemaphoreType.DMA,
        ],
    )
    def sc_edge(hs_hbm, ald_hbm, src_hbm, dst_hbm, zeros_hbm, out_hbm,
                acc_sh, ald_v, sidx_v, didx_v, rows_v, sem):
        cid = lax.axis_index("c")
        sid = lax.axis_index("s")
        wid = cid * 16 + sid

        pltpu.sync_copy(ald_hbm, ald_v)
        pltpu.sync_copy(zeros_hbm.at[pl.ds(sid * RP, RP)],
                        acc_sh.at[pl.ds(sid * RP, RP)])
        plsc.subcore_barrier()

        base = wid * EP
        iota = lax.iota(jnp.int32, 16)

        def chunk(ch, carry):
            off = base + ch * C
            pltpu.sync_copy(src_hbm.at[pl.ds(off, C)], sidx_v)
            pltpu.sync_copy(dst_hbm.at[pl.ds(off, C)], didx_v)
            pltpu.async_copy(hs_hbm.at[sidx_v], rows_v, sem).wait()
            for g in range(C // 16):
                rows16 = iota + g * 16
                d16 = didx_v[pl.ds(g * 16, 16)] * 2
                col0 = jnp.full((16,), Hc, jnp.int32)
                col1 = jnp.full((16,), Hc + 1, jnp.int32)
                a0 = (plsc.load_gather(rows_v, [rows16, col0])
                      + plsc.load_gather(ald_v, [d16]))
                a1 = (plsc.load_gather(rows_v, [rows16, col1])
                      + plsc.load_gather(ald_v, [d16 + 1]))
                a0 = jnp.where(a0 >= 0, a0, 0.2 * a0)
                a1 = jnp.where(a1 >= 0, a1, 0.2 * a1)
                ex0, ex1 = jnp.exp(a0), jnp.exp(a1)
                plsc.store_scatter(rows_v, [rows16, col0], ex0)
                plsc.store_scatter(rows_v, [rows16, col1], ex1)
                for j in range(16):
                    e = g * 16 + j
                    e0 = ex0[j]
                    e1 = ex1[j]
                    for k in range(Hc // 16):
                        sc = e0 if k < Hc // 32 else e1
                        rows_v[e, pl.ds(k * 16, 16)] = (
                            rows_v[e, pl.ds(k * 16, 16)] * sc)
            pltpu.sync_copy(rows_v, acc_sh.at[didx_v], add=True)
            return carry

        lax.fori_loop(0, NCH, chunk, 0)
        plsc.subcore_barrier()
        pltpu.sync_copy(acc_sh.at[pl.ds(sid * RP, RP)],
                        out_hbm.at[cid, pl.ds(sid * RP, RP)])

    return sc_edge


_GRID = 10
_B = N // _GRID


def _row_spec(cols):
    return pl.BlockSpec((_B, cols), lambda i: (i, 0))


def _full_spec(r, cols):
    return pl.BlockSpec((r, cols), lambda i: (0, 0))


def _logits(hs, a_ref, out_ref, base, c):
    for h in range(H):
        out_ref[:, base + h:base + h + 1] = jnp.sum(
            hs[:, h * c:(h + 1) * c] * a_ref[h:h + 1, :], axis=1, keepdims=True)


def _tc1_body(xp_ref, xa_ref,
              wsc, wdc, asc, adc, wsw, wdw, asw, adw, wsr, wdr, asr, adr,
              hsc_o, aldc_o, hsw_o, aldw_o, hsr_o, aldr_o):
    xp = xp_ref[...]
    xa = xa_ref[...]
    c = HID // H
    for xs, xd, ws, wd, a_s, a_d, hs_o, ald_o in (
            (xp, xp, wsc, wdc, asc, adc, hsc_o, aldc_o),
            (xa, xp, wsw, wdw, asw, adw, hsw_o, aldw_o),
            (xp, xa, wsr, wdr, asr, adr, hsr_o, aldr_o)):
        hs = jnp.dot(xs, ws[...], preferred_element_type=jnp.float32)
        hs_o[:, 0:HID] = hs
        _logits(hs, a_s, hs_o, HID, c)
        hs_o[:, HID + H:HID + 16] = jnp.zeros((_B, 16 - H), jnp.float32)
        hd = jnp.dot(xd, wd[...], preferred_element_type=jnp.float32)
        _logits(hd, a_d, ald_o, 0, c)


def _combine(acc_ref, hc):
    num = acc_ref[0, :, :] + acc_ref[1, :, :]
    c = hc // H
    d0 = num[:, hc:hc + 1] + 1e-16
    d1 = num[:, hc + 1:hc + 2] + 1e-16
    return jnp.concatenate([num[:, :c] / d0, num[:, c:hc] / d1], axis=1)


def _tc2_body(accc, accw, accr, b1c, b1w, b1r,
              wsc, wdc, asc, adc, wsw, wdw, asw, adw, wsr, wdr, asr, adr,
              hsc_o, aldc_o, hsw_o, aldw_o, hsr_o, aldr_o):
    hp = jax.nn.relu(_combine(accc, HID) + _combine(accw, HID)
                     + b1c[...] + b1w[...])
    ha = jax.nn.relu(_combine(accr, HID) + b1r[...])
    c = OUT // H
    for xs, xd, ws, wd, a_s, a_d, hs_o, ald_o in (
            (hp, hp, wsc, wdc, asc, adc, hsc_o, aldc_o),
            (ha, hp, wsw, wdw, asw, adw, hsw_o, aldw_o),
            (hp, ha, wsr, wdr, asr, adr, hsr_o, aldr_o)):
        hs = jnp.dot(xs, ws[...], preferred_element_type=jnp.float32)
        hs_o[:, 0:OUT] = hs
        _logits(hs, a_s, hs_o, OUT, c)
        hs_o[:, OUT + H:OUT + 16] = jnp.zeros((_B, 16 - H), jnp.float32)
        hd = jnp.dot(xd, wd[...], preferred_element_type=jnp.float32)
        _logits(hd, a_d, ald_o, 0, c)


def _tc3_body(accc, accw, accr, b2c, b2w, b2r, op_o, oa_o):
    op_o[...] = _combine(accc, OUT) + _combine(accw, OUT) + b2c[...] + b2w[...]
    oa_o[...] = _combine(accr, OUT) + b2r[...]


def _acc_spec(hcp):
    return pl.BlockSpec((2, _B, hcp), lambda i: (0, i, 0))


def kernel(x_paper, x_author, edge_index_cites, edge_index_writes,
           edge_index_rev_writes,
           l1_cites_Wsrc, l1_cites_Wdst, l1_cites_asrc, l1_cites_adst, l1_cites_b,
           l1_writes_Wsrc, l1_writes_Wdst, l1_writes_asrc, l1_writes_adst, l1_writes_b,
           l1_rev_writes_Wsrc, l1_rev_writes_Wdst, l1_rev_writes_asrc,
           l1_rev_writes_adst, l1_rev_writes_b,
           l2_cites_Wsrc, l2_cites_Wdst, l2_cites_asrc, l2_cites_adst, l2_cites_b,
           l2_writes_Wsrc, l2_writes_Wdst, l2_writes_asrc, l2_writes_adst, l2_writes_b,
           l2_rev_writes_Wsrc, l2_rev_writes_Wdst, l2_rev_writes_asrc,
           l2_rev_writes_adst, l2_rev_writes_b):
    f32 = jnp.float32
    ei_c = edge_index_cites.astype(jnp.int32)
    ei_w = edge_index_writes.astype(jnp.int32)
    ei_r = edge_index_rev_writes.astype(jnp.int32)

    row144 = jax.ShapeDtypeStruct((N, HID + 16), f32)
    al2 = jax.ShapeDtypeStruct((N, H), f32)
    tc1 = pl.pallas_call(
        _tc1_body,
        grid=(_GRID,),
        in_specs=[_row_spec(D), _row_spec(D)] + [
            s for _ in range(3) for s in
            (_full_spec(D, HID), _full_spec(D, HID),
             _full_spec(H, HID // H), _full_spec(H, HID // H))],
        out_specs=[s for _ in range(3) for s in
                   (_row_spec(HID + 16), _row_spec(H))],
        out_shape=[s for _ in range(3) for s in (row144, al2)],
    )
    (hs_c, ald_c, hs_w, ald_w, hs_r, ald_r) = tc1(
        x_paper, x_author,
        l1_cites_Wsrc, l1_cites_Wdst, l1_cites_asrc, l1_cites_adst,
        l1_writes_Wsrc, l1_writes_Wdst, l1_writes_asrc, l1_writes_adst,
        l1_rev_writes_Wsrc, l1_rev_writes_Wdst, l1_rev_writes_asrc,
        l1_rev_writes_adst)

    sc1 = _make_sc_edge(HID)
    zeros1 = jnp.zeros((N, HID + 16), f32)
    acc_c = sc1(hs_c, ald_c.reshape(-1), ei_c[0], ei_c[1], zeros1)
    acc_w = sc1(hs_w, ald_w.reshape(-1), ei_w[0], ei_w[1], zeros1)
    acc_r = sc1(hs_r, ald_r.reshape(-1), ei_r[0], ei_r[1], zeros1)

    row80 = jax.ShapeDtypeStruct((N, OUT + 16), f32)
    row64 = jax.ShapeDtypeStruct((N, OUT), f32)
    tc2 = pl.pallas_call(
        _tc2_body,
        grid=(_GRID,),
        in_specs=[_acc_spec(HID + 16)] * 3 + [_full_spec(1, HID)] * 3 + [
            s for _ in range(3) for s in
            (_full_spec(HID, OUT), _full_spec(HID, OUT),
             _full_spec(H, OUT // H), _full_spec(H, OUT // H))],
        out_specs=[s for _ in range(3) for s in
                   (_row_spec(OUT + 16), _row_spec(H))],
        out_shape=[s for _ in range(3) for s in (row80, al2)],
    )
    (hs2_c, ald2_c, hs2_w, ald2_w, hs2_r, ald2_r) = tc2(
        acc_c, acc_w, acc_r,
        l1_cites_b.reshape(1, HID), l1_writes_b.reshape(1, HID),
        l1_rev_writes_b.reshape(1, HID),
        l2_cites_Wsrc, l2_cites_Wdst, l2_cites_asrc, l2_cites_adst,
        l2_writes_Wsrc, l2_writes_Wdst, l2_writes_asrc, l2_writes_adst,
        l2_rev_writes_Wsrc, l2_rev_writes_Wdst, l2_rev_writes_asrc,
        l2_rev_writes_adst)

    sc2 = _make_sc_edge(OUT)
    zeros2 = jnp.zeros((N, OUT + 16), f32)
    acc2_c = sc2(hs2_c, ald2_c.reshape(-1), ei_c[0], ei_c[1], zeros2)
    acc2_w = sc2(hs2_w, ald2_w.reshape(-1), ei_w[0], ei_w[1], zeros2)
    acc2_r = sc2(hs2_r, ald2_r.reshape(-1), ei_r[0], ei_r[1], zeros2)

    tc3 = pl.pallas_call(
        _tc3_body,
        grid=(_GRID,),
        in_specs=[_acc_spec(OUT + 16)] * 3 + [_full_spec(1, OUT)] * 3,
        out_specs=[_row_spec(OUT), _row_spec(OUT)],
        out_shape=[row64, row64],
    )
    op, oa = tc3(acc2_c, acc2_w, acc2_r,
                 l2_cites_b.reshape(1, OUT), l2_writes_b.reshape(1, OUT),
                 l2_rev_writes_b.reshape(1, OUT))
    return op, oa
